# baseline (speedup 1.0000x reference)
import functools

import jax
import jax.numpy as jnp
from jax import lax
from jax.experimental import pallas as pl
from jax.experimental.pallas import tpu as pltpu
from jax.experimental.pallas import tpu_sc as plsc

ATOM_DIMS = (119, 5, 12, 12, 10, 6, 6, 2, 2)
NF = len(ATOM_DIMS)
H = 128
LANES = 16
NC, NS = 2, 16
NW = NC * NS
BLK = 64


def _encoder(xT_hbm, *rest):
    tabs = rest[:NF]
    out_hbm = rest[NF]
    idx_v = rest[NF + 1]
    bufs = rest[NF + 2:NF + 2 + NF]
    sem = rest[NF + 2 + NF]

    n = out_hbm.shape[0]
    nblocks = (n + BLK - 1) // BLK
    wid = lax.axis_index("s") * NC + lax.axis_index("c")
    nb_w = jnp.where(wid < nblocks, (nblocks - 1 - wid) // NW + 1, 0)

    def block_body(k, _):
        b = wid + k * NW
        start = jnp.minimum(b * BLK, n - BLK)
        idescs = [
            pltpu.async_copy(
                xT_hbm.at[pl.ds(t * n + start, BLK)], idx_v.at[t], sem)
            for t in range(NF)
        ]
        for d in idescs:
            d.wait()
        descs = [
            pltpu.async_copy(tabs[t].at[idx_v.at[t]], bufs[t], sem)
            for t in range(NF)
        ]
        for d in descs:
            d.wait()

        def row_body(r, _):
            for c in range(H // LANES):
                sl = pl.ds(c * LANES, LANES)
                acc = bufs[0][r, sl]
                for t in range(1, NF):
                    acc = acc + bufs[t][r, sl]
                bufs[0][r, sl] = acc
            return 0

        lax.fori_loop(0, BLK, row_body, 0, unroll=False)
        pltpu.sync_copy(bufs[0], out_hbm.at[pl.ds(start, BLK), :])
        return 0

    lax.fori_loop(0, nb_w, block_body, 0, unroll=False)


def kernel(x, tables):
    n = x.shape[0]
    xT = x.T.reshape(-1)

    mesh = plsc.VectorSubcoreMesh(
        core_axis_name="c", subcore_axis_name="s",
        num_cores=NC, num_subcores=NS,
    )
    run = functools.partial(
        pl.kernel,
        out_type=jax.ShapeDtypeStruct((n, H), jnp.float32),
        mesh=mesh,
        scratch_types=[
            pltpu.VMEM((NF, BLK), jnp.int32),
            *[pltpu.VMEM((BLK, H), jnp.float32) for _ in range(NF)],
            pltpu.SemaphoreType.DMA,
        ],
    )(_encoder)
    return run(xT, *tables)
